# Initial kernel scaffold; baseline (speedup 1.0000x reference)
import jax
import jax.numpy as jnp
from jax.experimental import pallas as pl


def kernel(indices, weights, labels, label_weights, size, tables, W1, b1, W2, b2, W3, b3, alpha1, alpha2, gamma0, beta0):
    raise NotImplementedError("write your pallas kernel here")


# speedup vs baseline: 3.1973x; 3.1973x over previous
import functools

import jax
import jax.numpy as jnp
import numpy as np
from jax import lax
from jax.experimental import pallas as pl
from jax.experimental.pallas import tpu as pltpu
from jax.experimental.pallas import tpu_sc as plsc

F = 26
V = 10000
K = 16
B = 4096
NE = F * K
D_EMB = F * NE
NPAIR = F * (F + 1) // 2
NEP = 512
HI = jax.lax.Precision.HIGHEST


_CH = 128


def _sc_gather(table2d, flat_idx):
    info = plsc.get_sparse_core_info()
    nw = info.num_cores * info.num_subcores
    n = F * B
    per_w = n // nw
    n_ch = per_w // _CH
    mesh = plsc.VectorSubcoreMesh(core_axis_name="c", subcore_axis_name="s")

    @functools.partial(
        pl.kernel,
        mesh=mesh,
        out_type=jax.ShapeDtypeStruct((n, NEP), jnp.float32),
        scratch_types=[
            pltpu.VMEM((_CH,), jnp.int32),
            pltpu.VMEM((_CH, NEP), jnp.float32),
            pltpu.SemaphoreType.DMA,
        ],
    )
    def k(table_hbm, idx_hbm, out_hbm, idx_v, rows_v, sem):
        wid = lax.axis_index("s") * info.num_cores + lax.axis_index("c")
        base = wid * per_w

        def body(c, carry):
            off = base + c * _CH
            pltpu.sync_copy(idx_hbm.at[pl.ds(off, _CH)], idx_v)
            pltpu.async_copy(table_hbm.at[idx_v], rows_v, sem).wait()
            pltpu.sync_copy(rows_v, out_hbm.at[pl.ds(off, _CH)])
            return carry

        lax.fori_loop(0, n_ch, body, 0)

    return k(table2d, flat_idx)


_BC = 128


def _c_body(g_ref, gt_ref, wt_ref, r_ref, s_ref,
            i3_ref, s1e_ref, s2e_ref, s1i_ref, s2i_ref):
    @pl.when(pl.program_id(0) == 0)
    def _init():
        s1e_ref[...] = jnp.zeros_like(s1e_ref)
        s2e_ref[...] = jnp.zeros_like(s2e_ref)
        s1i_ref[...] = jnp.zeros_like(s1i_ref)
        s2i_ref[...] = jnp.zeros_like(s2i_ref)

    g = g_ref[...][:, :, :NE]
    nrm = jnp.sqrt(jnp.sum(g * g, axis=2))
    st = jnp.minimum(1.0, 1.0 / jnp.maximum(nrm, 1e-12)) * wt_ref[...].T
    e = g * st[:, :, None]
    s1e_ref[...] += jnp.sum(e, axis=1)
    s2e_ref[...] += jnp.sum(e * e, axis=1)
    sexp = jnp.dot(st.T, r_ref[...], precision=HI)
    prod = e * gt_ref[...] * sexp[None, :, :]
    acc1 = []
    acc2 = []
    for i in range(F):
        it = jnp.dot(prod[i], s_ref[...], precision=HI)
        i3_ref[i] = it
        acc1.append(jnp.sum(it, axis=0))
        acc2.append(jnp.sum(it * it, axis=0))
    s1i_ref[...] += jnp.stack(acc1)
    s2i_ref[...] += jnp.stack(acc2)


def _pass_c(g3, gt3, w_t, r_sel, s_sel):
    grid = (B // _BC,)
    return pl.pallas_call(
        _c_body,
        grid=grid,
        in_specs=[
            pl.BlockSpec((F, _BC, NEP), lambda i: (0, i, 0)),
            pl.BlockSpec((F, _BC, NE), lambda i: (0, i, 0)),
            pl.BlockSpec((_BC, F), lambda i: (i, 0)),
            pl.BlockSpec((F, NE), lambda i: (0, 0)),
            pl.BlockSpec((NE, F), lambda i: (0, 0)),
        ],
        out_specs=[
            pl.BlockSpec((F, _BC, F), lambda i: (0, i, 0)),
            pl.BlockSpec((F, NE), lambda i: (0, 0)),
            pl.BlockSpec((F, NE), lambda i: (0, 0)),
            pl.BlockSpec((F, F), lambda i: (0, 0)),
            pl.BlockSpec((F, F), lambda i: (0, 0)),
        ],
        out_shape=[
            jax.ShapeDtypeStruct((F, B, F), jnp.float32),
            jax.ShapeDtypeStruct((F, NE), jnp.float32),
            jax.ShapeDtypeStruct((F, NE), jnp.float32),
            jax.ShapeDtypeStruct((F, F), jnp.float32),
            jax.ShapeDtypeStruct((F, F), jnp.float32),
        ],
        compiler_params=pltpu.CompilerParams(
            dimension_semantics=("arbitrary",)),
    )(g3, gt3, w_t, r_sel, s_sel)


def _d_body(g_ref, wt_ref, i3_ref, s1e_ref, s2e_ref, s1i_ref, s2i_ref,
            g0e_ref, b0e_ref, g0i_ref, b0i_ref, w1e_ref, w1i_ref, b1_ref,
            d1_ref):
    nf = float(B)
    m_e = s1e_ref[...] / nf
    v_e = jnp.maximum(s2e_ref[...] / nf - m_e * m_e, 0.0)
    inv_e = 1.0 / jnp.sqrt(v_e + 1e-5)
    a_e = g0e_ref[...] * inv_e
    c_e = b0e_ref[...] - m_e * a_e
    m_i = s1i_ref[...] / nf
    v_i = jnp.maximum(s2i_ref[...] / nf - m_i * m_i, 0.0)
    inv_i = 1.0 / jnp.sqrt(v_i + 1e-5)
    a_i = g0i_ref[...] * inv_i
    c_i = b0i_ref[...] - m_i * a_i

    g = g_ref[...][:, :, :NE]
    nrm = jnp.sqrt(jnp.sum(g * g, axis=2))
    st = jnp.minimum(1.0, 1.0 / jnp.maximum(nrm, 1e-12)) * wt_ref[...].T

    acc = jnp.zeros((_BC, 128), jnp.float32) + b1_ref[...]
    for f in range(F):
        ef = g[f] * st[f][:, None]
        xnf = ef * a_e[f][None, :] + c_e[f][None, :]
        acc += jnp.dot(xnf, w1e_ref[f], precision=HI)
    for i in range(F):
        xni = i3_ref[i] * a_i[i][None, :] + c_i[i][None, :]
        acc += jnp.dot(xni, w1i_ref[i], precision=HI)
    d1_ref[...] = acc


def _pass_d(g3, w_t, inter3, s1e, s2e, s1i, s2i,
            g0e, b0e, g0i, b0i, w1e_t, w1i_t, b1r):
    grid = (B // _BC,)
    return pl.pallas_call(
        _d_body,
        grid=grid,
        in_specs=[
            pl.BlockSpec((F, _BC, NEP), lambda i: (0, i, 0)),
            pl.BlockSpec((_BC, F), lambda i: (i, 0)),
            pl.BlockSpec((F, _BC, F), lambda i: (0, i, 0)),
            pl.BlockSpec((F, NE), lambda i: (0, 0)),
            pl.BlockSpec((F, NE), lambda i: (0, 0)),
            pl.BlockSpec((F, F), lambda i: (0, 0)),
            pl.BlockSpec((F, F), lambda i: (0, 0)),
            pl.BlockSpec((F, NE), lambda i: (0, 0)),
            pl.BlockSpec((F, NE), lambda i: (0, 0)),
            pl.BlockSpec((F, F), lambda i: (0, 0)),
            pl.BlockSpec((F, F), lambda i: (0, 0)),
            pl.BlockSpec((F, NE, 128), lambda i: (0, 0, 0)),
            pl.BlockSpec((F, F, 128), lambda i: (0, 0, 0)),
            pl.BlockSpec((1, 128), lambda i: (0, 0)),
        ],
        out_specs=pl.BlockSpec((_BC, 128), lambda i: (i, 0)),
        out_shape=jax.ShapeDtypeStruct((B, 128), jnp.float32),
        compiler_params=pltpu.CompilerParams(
            dimension_semantics=("arbitrary",)),
    )(g3, w_t, inter3, s1e, s2e, s1i, s2i, g0e, b0e, g0i, b0i,
      w1e_t, w1i_t, b1r)


def _e_body(d1_ref, al1_ref, w2t_ref, b2_ref, al2_ref, w3t_ref, b3_ref,
            lab_ref, lw_ref, dout_ref, s_ref, loss_ref):
    d1 = d1_ref[...]
    m1 = jnp.mean(d1, axis=0, keepdims=True)
    v1 = jnp.mean((d1 - m1) * (d1 - m1), axis=0, keepdims=True)
    xn1 = (d1 - m1) / jnp.sqrt(v1 + 1e-8)
    p1 = 1.0 / (1.0 + jnp.exp(-xn1))
    h1 = (p1 + (1.0 - p1) * al1_ref[...]) * d1
    d2 = jnp.dot(h1, w2t_ref[...], precision=HI) + b2_ref[...]
    m2 = jnp.mean(d2, axis=0, keepdims=True)
    v2 = jnp.mean((d2 - m2) * (d2 - m2), axis=0, keepdims=True)
    xn2 = (d2 - m2) / jnp.sqrt(v2 + 1e-8)
    p2 = 1.0 / (1.0 + jnp.exp(-xn2))
    dout = (p2 + (1.0 - p2) * al2_ref[...]) * d2
    s = jnp.dot(dout, w3t_ref[...], precision=HI) + b3_ref[...]
    ls = lw_ref[...] * (jnp.maximum(s, 0.0) - s * lab_ref[...]
                        + jnp.log(1.0 + jnp.exp(-jnp.abs(s))))
    dout_ref[...] = dout
    s_ref[...] = s
    loss_ref[...] = jnp.sum(ls, keepdims=True).reshape(1, 1)


def _pass_e(d1, al1, w2t, b2r, al2, w3t, b3r, lab, lw):
    return pl.pallas_call(
        _e_body,
        out_shape=[
            jax.ShapeDtypeStruct((B, 64), jnp.float32),
            jax.ShapeDtypeStruct((B, 1), jnp.float32),
            jax.ShapeDtypeStruct((1, 1), jnp.float32),
        ],
    )(d1, al1, w2t, b2r, al2, w3t, b3r, lab, lw)


def _pair_maps():
    iu, ju = np.triu_indices(F)
    pid = np.zeros((F, F), np.int32)
    pid[iu, ju] = np.arange(NPAIR)
    pid[ju, iu] = np.arange(NPAIR)
    halve = np.where(np.eye(F, dtype=bool), 1.0, 0.5).astype(np.float32)
    return pid, halve


_PID, _HALVE = _pair_maps()

_R_SEL = np.zeros((F, NE), np.float32)
for _j in range(F):
    _R_SEL[_j, 16 * _j:16 * (_j + 1)] = 1.0
_S_SEL = _R_SEL.T.copy()


def kernel(indices, weights, labels, label_weights, size, tables,
           W1, b1, W2, b2, W3, b3, alpha1, alpha2, gamma0, beta0):
    table2d = jnp.pad(tables.reshape(F * V, NE), ((0, 0), (0, NEP - NE)))
    idx = indices.astype(jnp.int32)
    flat_idx = (idx + (jnp.arange(F, dtype=jnp.int32) * V)[:, None]).reshape(-1)

    g = _sc_gather(table2d, flat_idx)
    g3 = g.reshape(F, B, NEP)
    gt3 = jnp.transpose(
        g3[:, :, :NE].reshape(F, B, F, K), (2, 1, 0, 3)).reshape(F, B, NE)
    w_t = weights.T

    w1e_t = jnp.transpose(W1[:, :D_EMB].reshape(128, F, NE), (1, 2, 0))
    w1i = W1[:, D_EMB:][:, _PID] * _HALVE[None]
    w1i_t = jnp.transpose(w1i, (1, 2, 0))
    g0e = gamma0[:D_EMB].reshape(F, NE)
    b0e = beta0[:D_EMB].reshape(F, NE)
    g0i = gamma0[D_EMB:][_PID]
    b0i = beta0[D_EMB:][_PID]
    r_sel = jnp.asarray(_R_SEL)
    s_sel = jnp.asarray(_S_SEL)

    inter3, s1e, s2e, s1i, s2i = _pass_c(g3, gt3, w_t, r_sel, s_sel)
    d1 = _pass_d(g3, w_t, inter3, s1e, s2e, s1i, s2i,
                 g0e, b0e, g0i, b0i, w1e_t, w1i_t, b1.reshape(1, 128))
    dout, s_col, loss = _pass_e(
        d1, alpha1.reshape(1, 128), W2.T, b2.reshape(1, 64),
        alpha2.reshape(1, 64), W3.T, b3.reshape(1, 1),
        labels.reshape(B, 1), label_weights.reshape(B, 1))

    final_loss = loss.reshape(()) / size
    return (final_loss, s_col.reshape(-1), dout)


# speedup vs baseline: 3.2041x; 1.0021x over previous
import functools

import jax
import jax.numpy as jnp
import numpy as np
from jax import lax
from jax.experimental import pallas as pl
from jax.experimental.pallas import tpu as pltpu
from jax.experimental.pallas import tpu_sc as plsc

F = 26
V = 10000
K = 16
B = 4096
NE = F * K
D_EMB = F * NE
NPAIR = F * (F + 1) // 2
NEP = 512
HI = jax.lax.Precision.HIGHEST


_CH = 128


def _sc_gather(table2d, flat_idx):
    info = plsc.get_sparse_core_info()
    nw = info.num_cores * info.num_subcores
    n = F * B
    per_w = n // nw
    ch = 64
    n_pair = per_w // (2 * ch)
    mesh = plsc.VectorSubcoreMesh(core_axis_name="c", subcore_axis_name="s")

    @functools.partial(
        pl.kernel,
        mesh=mesh,
        out_type=jax.ShapeDtypeStruct((n, NEP), jnp.float32),
        scratch_types=[
            pltpu.VMEM((per_w,), jnp.int32),
            pltpu.VMEM((ch, NEP), jnp.float32),
            pltpu.VMEM((ch, NEP), jnp.float32),
            pltpu.SemaphoreType.DMA,
            pltpu.SemaphoreType.DMA,
            pltpu.SemaphoreType.DMA,
            pltpu.SemaphoreType.DMA,
        ],
    )
    def k(table_hbm, idx_hbm, out_hbm, idx_v, rows0, rows1, gs0, gs1,
          ws0, ws1):
        wid = lax.axis_index("s") * info.num_cores + lax.axis_index("c")
        base = wid * per_w
        pltpu.sync_copy(idx_hbm.at[pl.ds(base, per_w)], idx_v)

        def body(p, carry):
            c0 = 2 * p * ch
            c1 = c0 + ch

            @pl.when(p > 0)
            def _drain_prev():
                pltpu.make_async_copy(
                    rows0, out_hbm.at[pl.ds(base, ch)], ws0).wait()
                pltpu.make_async_copy(
                    rows1, out_hbm.at[pl.ds(base, ch)], ws1).wait()

            g0 = pltpu.async_copy(
                table_hbm.at[idx_v.at[pl.ds(c0, ch)]], rows0, gs0)
            g1 = pltpu.async_copy(
                table_hbm.at[idx_v.at[pl.ds(c1, ch)]], rows1, gs1)
            g0.wait()
            pltpu.async_copy(rows0, out_hbm.at[pl.ds(base + c0, ch)], ws0)
            g1.wait()
            pltpu.async_copy(rows1, out_hbm.at[pl.ds(base + c1, ch)], ws1)
            return carry

        lax.fori_loop(0, n_pair, body, 0)
        pltpu.make_async_copy(rows0, out_hbm.at[pl.ds(base, ch)], ws0).wait()
        pltpu.make_async_copy(rows1, out_hbm.at[pl.ds(base, ch)], ws1).wait()

    return k(table2d, flat_idx)


_BC = 128


def _c_body(g_ref, gt_ref, wt_ref, r_ref, s_ref,
            i3_ref, s1e_ref, s2e_ref, s1i_ref, s2i_ref):
    @pl.when(pl.program_id(0) == 0)
    def _init():
        s1e_ref[...] = jnp.zeros_like(s1e_ref)
        s2e_ref[...] = jnp.zeros_like(s2e_ref)
        s1i_ref[...] = jnp.zeros_like(s1i_ref)
        s2i_ref[...] = jnp.zeros_like(s2i_ref)

    g = g_ref[...][:, :, :NE]
    nrm = jnp.sqrt(jnp.sum(g * g, axis=2))
    st = jnp.minimum(1.0, 1.0 / jnp.maximum(nrm, 1e-12)) * wt_ref[...].T
    e = g * st[:, :, None]
    s1e_ref[...] += jnp.sum(e, axis=1)
    s2e_ref[...] += jnp.sum(e * e, axis=1)
    sexp = jnp.dot(st.T, r_ref[...], precision=HI)
    prod = e * gt_ref[...] * sexp[None, :, :]
    acc1 = []
    acc2 = []
    for i in range(F):
        it = jnp.dot(prod[i], s_ref[...], precision=HI)
        i3_ref[i] = it
        acc1.append(jnp.sum(it, axis=0))
        acc2.append(jnp.sum(it * it, axis=0))
    s1i_ref[...] += jnp.stack(acc1)
    s2i_ref[...] += jnp.stack(acc2)


def _pass_c(g3, gt3, w_t, r_sel, s_sel):
    grid = (B // _BC,)
    return pl.pallas_call(
        _c_body,
        grid=grid,
        in_specs=[
            pl.BlockSpec((F, _BC, NEP), lambda i: (0, i, 0)),
            pl.BlockSpec((F, _BC, NE), lambda i: (0, i, 0)),
            pl.BlockSpec((_BC, F), lambda i: (i, 0)),
            pl.BlockSpec((F, NE), lambda i: (0, 0)),
            pl.BlockSpec((NE, F), lambda i: (0, 0)),
        ],
        out_specs=[
            pl.BlockSpec((F, _BC, F), lambda i: (0, i, 0)),
            pl.BlockSpec((F, NE), lambda i: (0, 0)),
            pl.BlockSpec((F, NE), lambda i: (0, 0)),
            pl.BlockSpec((F, F), lambda i: (0, 0)),
            pl.BlockSpec((F, F), lambda i: (0, 0)),
        ],
        out_shape=[
            jax.ShapeDtypeStruct((F, B, F), jnp.float32),
            jax.ShapeDtypeStruct((F, NE), jnp.float32),
            jax.ShapeDtypeStruct((F, NE), jnp.float32),
            jax.ShapeDtypeStruct((F, F), jnp.float32),
            jax.ShapeDtypeStruct((F, F), jnp.float32),
        ],
        compiler_params=pltpu.CompilerParams(
            dimension_semantics=("arbitrary",)),
    )(g3, gt3, w_t, r_sel, s_sel)


def _d_body(g_ref, wt_ref, i3_ref, s1e_ref, s2e_ref, s1i_ref, s2i_ref,
            g0e_ref, b0e_ref, g0i_ref, b0i_ref, w1e_ref, w1i_ref, b1_ref,
            d1_ref):
    nf = float(B)
    m_e = s1e_ref[...] / nf
    v_e = jnp.maximum(s2e_ref[...] / nf - m_e * m_e, 0.0)
    inv_e = 1.0 / jnp.sqrt(v_e + 1e-5)
    a_e = g0e_ref[...] * inv_e
    c_e = b0e_ref[...] - m_e * a_e
    m_i = s1i_ref[...] / nf
    v_i = jnp.maximum(s2i_ref[...] / nf - m_i * m_i, 0.0)
    inv_i = 1.0 / jnp.sqrt(v_i + 1e-5)
    a_i = g0i_ref[...] * inv_i
    c_i = b0i_ref[...] - m_i * a_i

    g = g_ref[...][:, :, :NE]
    nrm = jnp.sqrt(jnp.sum(g * g, axis=2))
    st = jnp.minimum(1.0, 1.0 / jnp.maximum(nrm, 1e-12)) * wt_ref[...].T

    acc = jnp.zeros((_BC, 128), jnp.float32) + b1_ref[...]
    for f in range(F):
        ef = g[f] * st[f][:, None]
        xnf = ef * a_e[f][None, :] + c_e[f][None, :]
        acc += jnp.dot(xnf, w1e_ref[f], precision=HI)
    for i in range(F):
        xni = i3_ref[i] * a_i[i][None, :] + c_i[i][None, :]
        acc += jnp.dot(xni, w1i_ref[i], precision=HI)
    d1_ref[...] = acc


def _pass_d(g3, w_t, inter3, s1e, s2e, s1i, s2i,
            g0e, b0e, g0i, b0i, w1e_t, w1i_t, b1r):
    grid = (B // _BC,)
    return pl.pallas_call(
        _d_body,
        grid=grid,
        in_specs=[
            pl.BlockSpec((F, _BC, NEP), lambda i: (0, i, 0)),
            pl.BlockSpec((_BC, F), lambda i: (i, 0)),
            pl.BlockSpec((F, _BC, F), lambda i: (0, i, 0)),
            pl.BlockSpec((F, NE), lambda i: (0, 0)),
            pl.BlockSpec((F, NE), lambda i: (0, 0)),
            pl.BlockSpec((F, F), lambda i: (0, 0)),
            pl.BlockSpec((F, F), lambda i: (0, 0)),
            pl.BlockSpec((F, NE), lambda i: (0, 0)),
            pl.BlockSpec((F, NE), lambda i: (0, 0)),
            pl.BlockSpec((F, F), lambda i: (0, 0)),
            pl.BlockSpec((F, F), lambda i: (0, 0)),
            pl.BlockSpec((F, NE, 128), lambda i: (0, 0, 0)),
            pl.BlockSpec((F, F, 128), lambda i: (0, 0, 0)),
            pl.BlockSpec((1, 128), lambda i: (0, 0)),
        ],
        out_specs=pl.BlockSpec((_BC, 128), lambda i: (i, 0)),
        out_shape=jax.ShapeDtypeStruct((B, 128), jnp.float32),
        compiler_params=pltpu.CompilerParams(
            dimension_semantics=("arbitrary",)),
    )(g3, w_t, inter3, s1e, s2e, s1i, s2i, g0e, b0e, g0i, b0i,
      w1e_t, w1i_t, b1r)


def _e_body(d1_ref, al1_ref, w2t_ref, b2_ref, al2_ref, w3t_ref, b3_ref,
            lab_ref, lw_ref, dout_ref, s_ref, loss_ref):
    d1 = d1_ref[...]
    m1 = jnp.mean(d1, axis=0, keepdims=True)
    v1 = jnp.mean((d1 - m1) * (d1 - m1), axis=0, keepdims=True)
    xn1 = (d1 - m1) / jnp.sqrt(v1 + 1e-8)
    p1 = 1.0 / (1.0 + jnp.exp(-xn1))
    h1 = (p1 + (1.0 - p1) * al1_ref[...]) * d1
    d2 = jnp.dot(h1, w2t_ref[...], precision=HI) + b2_ref[...]
    m2 = jnp.mean(d2, axis=0, keepdims=True)
    v2 = jnp.mean((d2 - m2) * (d2 - m2), axis=0, keepdims=True)
    xn2 = (d2 - m2) / jnp.sqrt(v2 + 1e-8)
    p2 = 1.0 / (1.0 + jnp.exp(-xn2))
    dout = (p2 + (1.0 - p2) * al2_ref[...]) * d2
    s = jnp.dot(dout, w3t_ref[...], precision=HI) + b3_ref[...]
    ls = lw_ref[...] * (jnp.maximum(s, 0.0) - s * lab_ref[...]
                        + jnp.log(1.0 + jnp.exp(-jnp.abs(s))))
    dout_ref[...] = dout
    s_ref[...] = s
    loss_ref[...] = jnp.sum(ls, keepdims=True).reshape(1, 1)


def _pass_e(d1, al1, w2t, b2r, al2, w3t, b3r, lab, lw):
    return pl.pallas_call(
        _e_body,
        out_shape=[
            jax.ShapeDtypeStruct((B, 64), jnp.float32),
            jax.ShapeDtypeStruct((B, 1), jnp.float32),
            jax.ShapeDtypeStruct((1, 1), jnp.float32),
        ],
    )(d1, al1, w2t, b2r, al2, w3t, b3r, lab, lw)


def _pair_maps():
    iu, ju = np.triu_indices(F)
    pid = np.zeros((F, F), np.int32)
    pid[iu, ju] = np.arange(NPAIR)
    pid[ju, iu] = np.arange(NPAIR)
    halve = np.where(np.eye(F, dtype=bool), 1.0, 0.5).astype(np.float32)
    return pid, halve


_PID, _HALVE = _pair_maps()

_R_SEL = np.zeros((F, NE), np.float32)
for _j in range(F):
    _R_SEL[_j, 16 * _j:16 * (_j + 1)] = 1.0
_S_SEL = _R_SEL.T.copy()


def kernel(indices, weights, labels, label_weights, size, tables,
           W1, b1, W2, b2, W3, b3, alpha1, alpha2, gamma0, beta0):
    table2d = jnp.pad(tables.reshape(F * V, NE), ((0, 0), (0, NEP - NE)))
    idx = indices.astype(jnp.int32)
    flat_idx = (idx + (jnp.arange(F, dtype=jnp.int32) * V)[:, None]).reshape(-1)

    g = _sc_gather(table2d, flat_idx)
    g3 = g.reshape(F, B, NEP)
    gt3 = jnp.transpose(
        g3[:, :, :NE].reshape(F, B, F, K), (2, 1, 0, 3)).reshape(F, B, NE)
    w_t = weights.T

    w1e_t = jnp.transpose(W1[:, :D_EMB].reshape(128, F, NE), (1, 2, 0))
    w1i = W1[:, D_EMB:][:, _PID] * _HALVE[None]
    w1i_t = jnp.transpose(w1i, (1, 2, 0))
    g0e = gamma0[:D_EMB].reshape(F, NE)
    b0e = beta0[:D_EMB].reshape(F, NE)
    g0i = gamma0[D_EMB:][_PID]
    b0i = beta0[D_EMB:][_PID]
    r_sel = jnp.asarray(_R_SEL)
    s_sel = jnp.asarray(_S_SEL)

    inter3, s1e, s2e, s1i, s2i = _pass_c(g3, gt3, w_t, r_sel, s_sel)
    d1 = _pass_d(g3, w_t, inter3, s1e, s2e, s1i, s2i,
                 g0e, b0e, g0i, b0i, w1e_t, w1i_t, b1.reshape(1, 128))
    dout, s_col, loss = _pass_e(
        d1, alpha1.reshape(1, 128), W2.T, b2.reshape(1, 64),
        alpha2.reshape(1, 64), W3.T, b3.reshape(1, 1),
        labels.reshape(B, 1), label_weights.reshape(B, 1))

    final_loss = loss.reshape(()) / size
    return (final_loss, s_col.reshape(-1), dout)
